# Initial kernel scaffold; baseline (speedup 1.0000x reference)
import jax
import jax.numpy as jnp
from jax.experimental import pallas as pl


def kernel(h_node, h_edge, edge_index, params):
    raise NotImplementedError("write your pallas kernel here")


# speedup vs baseline: 20.6673x; 20.6673x over previous
import functools

import numpy as np
import jax
import jax.numpy as jnp
from jax import lax
from jax.experimental import pallas as pl
from jax.experimental.pallas import tpu as pltpu
from jax.experimental.pallas import tpu_sc as plsc

N = 10000
E = 320000
D = 128
H = 8
DH = D // H
SCALE = DH ** -0.5

NC = 2
NS = 16
NW = NC * NS
PER_W = E // NW
CB = 80
NCH = PER_W // CB
NPAD = 10240
ROWS_PER_SUB = NPAD // NS

EB = 2000
GE = E // EB
NB = 400
GN = N // NB

_EXP_NP = np.kron(np.eye(H, dtype=np.float32), np.ones((1, DH), np.float32))

def _ln(x, g, b, eps=1e-5):
    m = jnp.mean(x, axis=-1, keepdims=True)
    v = jnp.mean((x - m) ** 2, axis=-1, keepdims=True)
    return (x - m) / jnp.sqrt(v + eps) * g + b


def _gelu(x):
    return x * 0.5 * (1.0 + lax.erf(x * np.float32(1.0 / np.sqrt(2.0))))


def _dot(a, b):
    return jnp.dot(a, b, preferred_element_type=jnp.float32)


def _tc_qkv_body(x_ref, wq, bq, wk, bk, wv, bv, q_out, k_out, v_out):
    x = x_ref[...]
    q_out[...] = _dot(x, wq[...]) + bq[...]
    k_out[...] = _dot(x, wk[...]) + bk[...]
    v_out[...] = _dot(x, wv[...]) + bv[...]


def _tc_eb_body(he_ref, w, b, eb_out):
    eb_out[...] = _dot(he_ref[...], w[...]) + b[...]


def _tc_msg_body(qd, ks, vs, eb, hs, expm, msg_out, exx_out):
    s = _dot(qd[...] * ks[...], hs[...]) * SCALE + eb[...]
    exx = _dot(jnp.exp(s), expm[...])
    exx_out[...] = exx
    msg_out[...] = vs[...] * exx


def _tc_node_body(hnode, on0, on1, den0, den1,
                  wo, bo, g1, b1, wf1, bf1, wf2, bf2, g2, b2,
                  w1a, b1u, w1b, hn_out, a_out, b_out):
    agg = (on0[...] + on1[...]) / (den0[...] + den1[...] + 1e-16)
    out_node = _dot(agg, wo[...]) + bo[...]
    h1 = _ln(hnode[...] + out_node, g1[...], b1[...])
    ff = _dot(_gelu(_dot(h1, wf1[...]) + bf1[...]), wf2[...]) + bf2[...]
    hn = _ln(h1 + ff, g2[...], b2[...])
    hn_out[...] = hn
    a_out[...] = _dot(hn, w1a[...]) + b1u[...]
    b_out[...] = _dot(hn, w1b[...])


def _tc_edge_body(an, bn, he, w1c, w2, b2, ge, be, he_out):
    t = an[...] + bn[...] + _dot(he[...], w1c[...])
    hen = _dot(_gelu(t), w2[...]) + b2[...]
    he_out[...] = _ln(he[...] + hen, ge[...], be[...])


@functools.cache
def _sc_kernels():
    mesh = plsc.VectorSubcoreMesh(core_axis_name="c", subcore_axis_name="s")

    @functools.partial(
        pl.kernel,
        mesh=mesh,
        out_type=[jax.ShapeDtypeStruct((E, D), jnp.float32)] * 3,
        scratch_types=[
            pltpu.VMEM((CB,), jnp.int32),
            pltpu.VMEM((CB,), jnp.int32),
            pltpu.VMEM((CB, D), jnp.float32),
            pltpu.VMEM((CB, D), jnp.float32),
            pltpu.VMEM((CB, D), jnp.float32),
            pltpu.SemaphoreType.DMA,
            pltpu.SemaphoreType.DMA,
        ],
    )
    def gather_qkv(q_hbm, k_hbm, v_hbm, src_hbm, dst_hbm, qd_out, ks_out,
                   vs_out, idx_s, idx_d, bq, bk, bv, sem_g, sem_w):
        wid = lax.axis_index("s") * NC + lax.axis_index("c")
        base = wid * PER_W

        def chunk(c, carry):
            off = base + c * CB
            pltpu.sync_copy(src_hbm.at[pl.ds(off, CB)], idx_s)
            pltpu.sync_copy(dst_hbm.at[pl.ds(off, CB)], idx_d)
            cq = pltpu.async_copy(q_hbm.at[idx_d], bq, sem_g)
            ck = pltpu.async_copy(k_hbm.at[idx_s], bk, sem_g)
            cv = pltpu.async_copy(v_hbm.at[idx_s], bv, sem_g)
            cq.wait()
            ck.wait()
            cv.wait()
            w1 = pltpu.async_copy(bq, qd_out.at[pl.ds(off, CB)], sem_w)
            w2 = pltpu.async_copy(bk, ks_out.at[pl.ds(off, CB)], sem_w)
            w3 = pltpu.async_copy(bv, vs_out.at[pl.ds(off, CB)], sem_w)
            w1.wait()
            w2.wait()
            w3.wait()
            return carry

        lax.fori_loop(0, NCH, chunk, 0)

    @functools.partial(
        pl.kernel,
        mesh=mesh,
        out_type=[jax.ShapeDtypeStruct((E, D), jnp.float32)] * 2,
        scratch_types=[
            pltpu.VMEM((CB,), jnp.int32),
            pltpu.VMEM((CB,), jnp.int32),
            pltpu.VMEM((CB, D), jnp.float32),
            pltpu.VMEM((CB, D), jnp.float32),
            pltpu.SemaphoreType.DMA,
            pltpu.SemaphoreType.DMA,
        ],
    )
    def gather_ab(a_hbm, b_hbm, src_hbm, dst_hbm, an_out, bn_out,
                  idx_s, idx_d, ba, bb, sem_g, sem_w):
        wid = lax.axis_index("s") * NC + lax.axis_index("c")
        base = wid * PER_W

        def chunk(c, carry):
            off = base + c * CB
            pltpu.sync_copy(src_hbm.at[pl.ds(off, CB)], idx_s)
            pltpu.sync_copy(dst_hbm.at[pl.ds(off, CB)], idx_d)
            ca = pltpu.async_copy(a_hbm.at[idx_s], ba, sem_g)
            cb = pltpu.async_copy(b_hbm.at[idx_d], bb, sem_g)
            ca.wait()
            cb.wait()
            w1 = pltpu.async_copy(ba, an_out.at[pl.ds(off, CB)], sem_w)
            w2 = pltpu.async_copy(bb, bn_out.at[pl.ds(off, CB)], sem_w)
            w1.wait()
            w2.wait()
            return carry

        lax.fori_loop(0, NCH, chunk, 0)

    @functools.partial(
        pl.kernel,
        mesh=mesh,
        out_type=[jax.ShapeDtypeStruct((NC, NPAD, D), jnp.float32)] * 2,
        scratch_types=[
            pltpu.VMEM((CB,), jnp.int32),
            pltpu.VMEM((CB, D), jnp.float32),
            pltpu.VMEM_SHARED((NPAD, D), jnp.float32),
        ],
    )
    def scatter(dst_hbm, msg_hbm, exx_hbm, zero_hbm, on_out, den_out,
                idx_d, buf, sh):
        cid = lax.axis_index("c")
        sid = lax.axis_index("s")
        wid = sid * NC + cid
        r0 = sid * ROWS_PER_SUB
        base = wid * PER_W

        def phase(payload_hbm, out_hbm):
            pltpu.sync_copy(zero_hbm.at[pl.ds(r0, ROWS_PER_SUB)],
                            sh.at[pl.ds(r0, ROWS_PER_SUB)])
            plsc.subcore_barrier()

            def chunk(c, carry):
                off = base + c * CB
                pltpu.sync_copy(dst_hbm.at[pl.ds(off, CB)], idx_d)
                pltpu.sync_copy(payload_hbm.at[pl.ds(off, CB)], buf)
                pltpu.sync_copy(buf, sh.at[idx_d], add=True)
                return carry

            lax.fori_loop(0, NCH, chunk, 0)
            plsc.subcore_barrier()
            pltpu.sync_copy(sh.at[pl.ds(r0, ROWS_PER_SUB)],
                            out_hbm.at[cid, pl.ds(r0, ROWS_PER_SUB)])
            plsc.subcore_barrier()

        phase(msg_hbm, on_out)
        phase(exx_hbm, den_out)

    return gather_qkv, gather_ab, scatter


def kernel(h_node, h_edge, edge_index, params):
    p = params
    _sc_gather_qkv, _sc_gather_ab, _sc_scatter = _sc_kernels()
    ei = edge_index.astype(jnp.int32)
    e_src, e_dst = ei[0], ei[1]
    expm = jnp.asarray(_EXP_NP)
    hs = jnp.asarray(_EXP_NP.T)
    r2 = lambda t: t.reshape(1, -1)

    q, k, v = pl.pallas_call(
        _tc_qkv_body,
        out_shape=[jax.ShapeDtypeStruct((N, D), jnp.float32)] * 3,
    )(h_node, p["q"]["W"], r2(p["q"]["b"]), p["k"]["W"], r2(p["k"]["b"]),
      p["v"]["W"], r2(p["v"]["b"]))

    eb = pl.pallas_call(
        _tc_eb_body,
        grid=(GE,),
        in_specs=[pl.BlockSpec((EB, D), lambda i: (i, 0)),
                  pl.BlockSpec((D, H), lambda i: (0, 0)),
                  pl.BlockSpec((1, H), lambda i: (0, 0))],
        out_specs=pl.BlockSpec((EB, H), lambda i: (i, 0)),
        out_shape=jax.ShapeDtypeStruct((E, H), jnp.float32),
    )(h_edge, p["eb"]["W"], r2(p["eb"]["b"]))

    qd, ks, vs = _sc_gather_qkv(q, k, v, e_src, e_dst)

    comb = pl.pallas_call(
        _tc_msg_body,
        grid=(GE,),
        in_specs=[pl.BlockSpec((EB, D), lambda i: (i, 0)),
                  pl.BlockSpec((EB, D), lambda i: (i, 0)),
                  pl.BlockSpec((EB, D), lambda i: (i, 0)),
                  pl.BlockSpec((EB, H), lambda i: (i, 0)),
                  pl.BlockSpec((D, H), lambda i: (0, 0)),
                  pl.BlockSpec((H, D), lambda i: (0, 0))],
        out_specs=[pl.BlockSpec((EB, D), lambda i: (i, 0))] * 2,
        out_shape=[jax.ShapeDtypeStruct((E, D), jnp.float32)] * 2,
    )(qd, ks, vs, eb, hs, expm)
    msg, exx = comb

    zero = jnp.zeros((NPAD, D), jnp.float32)
    on_p, den_p = _sc_scatter(e_dst, msg, exx, zero)

    wspec = lambda shp: pl.BlockSpec(shp, lambda i: (0, 0))
    hn, a_tab, b_tab = pl.pallas_call(
        _tc_node_body,
        grid=(GN,),
        in_specs=[pl.BlockSpec((NB, D), lambda i: (i, 0))] * 5 +
                 [wspec((D, D)), wspec((1, D)),
                  wspec((1, D)), wspec((1, D)),
                  wspec((D, 2 * D)), wspec((1, 2 * D)),
                  wspec((2 * D, D)), wspec((1, D)),
                  wspec((1, D)), wspec((1, D)),
                  wspec((D, D)), wspec((1, D)), wspec((D, D))],
        out_specs=[pl.BlockSpec((NB, D), lambda i: (i, 0))] * 3,
        out_shape=[jax.ShapeDtypeStruct((N, D), jnp.float32)] * 3,
    )(h_node, on_p[0], on_p[1], den_p[0], den_p[1],
      p["o"]["W"], r2(p["o"]["b"]),
      r2(p["ln1"]["g"]), r2(p["ln1"]["b"]),
      p["ffn1"]["W"], r2(p["ffn1"]["b"]),
      p["ffn2"]["W"], r2(p["ffn2"]["b"]),
      r2(p["ln2"]["g"]), r2(p["ln2"]["b"]),
      p["eu1"]["W"][:D], r2(p["eu1"]["b"]), p["eu1"]["W"][D:2 * D])

    an, bn = _sc_gather_ab(a_tab, b_tab, e_src, e_dst)

    he = pl.pallas_call(
        _tc_edge_body,
        grid=(GE,),
        in_specs=[pl.BlockSpec((EB, D), lambda i: (i, 0)),
                  pl.BlockSpec((EB, D), lambda i: (i, 0)),
                  pl.BlockSpec((EB, D), lambda i: (i, 0)),
                  wspec((D, D)), wspec((D, D)), wspec((1, D)),
                  wspec((1, D)), wspec((1, D))],
        out_specs=pl.BlockSpec((EB, D), lambda i: (i, 0)),
        out_shape=jax.ShapeDtypeStruct((E, D), jnp.float32),
    )(an, bn, h_edge, p["eu1"]["W"][2 * D:], p["eu2"]["W"], r2(p["eu2"]["b"]),
      r2(p["lne"]["g"]), r2(p["lne"]["b"]))

    return hn, he


# speedup vs baseline: 28.1441x; 1.3618x over previous
import functools

import numpy as np
import jax
import jax.numpy as jnp
from jax import lax
from jax.experimental import pallas as pl
from jax.experimental.pallas import tpu as pltpu
from jax.experimental.pallas import tpu_sc as plsc

N = 10000
E = 320000
D = 128
H = 8
DH = D // H
SCALE = DH ** -0.5

NC = 2
NS = 16
NW = NC * NS
PER_W = E // NW
CB = 80
NCH = PER_W // CB
NPAD = 10240
ROWS_PER_SUB = NPAD // NS

EB = 2000
GE = E // EB
NB = 400
GN = N // NB

_EXP_NP = np.kron(np.eye(H, dtype=np.float32), np.ones((1, DH), np.float32))

def _ln(x, g, b, eps=1e-5):
    m = jnp.mean(x, axis=-1, keepdims=True)
    v = jnp.mean((x - m) ** 2, axis=-1, keepdims=True)
    return (x - m) / jnp.sqrt(v + eps) * g + b


def _gelu(x):
    return x * 0.5 * (1.0 + lax.erf(x * np.float32(1.0 / np.sqrt(2.0))))


def _dot(a, b):
    return jnp.dot(a, b, preferred_element_type=jnp.float32)


def _tc_qkv_body(x_ref, wq, bq, wk, bk, wv, bv, q_out, k_out, v_out):
    x = x_ref[...]
    q_out[...] = _dot(x, wq[...]) + bq[...]
    k_out[...] = _dot(x, wk[...]) + bk[...]
    v_out[...] = _dot(x, wv[...]) + bv[...]


def _tc_eb_body(he_ref, w, b, eb_out):
    eb_out[...] = _dot(he_ref[...], w[...]) + b[...]


def _tc_msg_body(qd, ks, vs, eb, hs, expm, msg_out, exx_out):
    s = _dot(qd[...] * ks[...], hs[...]) * SCALE + eb[...]
    exx = _dot(jnp.exp(s), expm[...])
    exx_out[...] = exx
    msg_out[...] = vs[...] * exx


def _tc_node_body(hnode, on0, on1, den0, den1,
                  wo, bo, g1, b1, wf1, bf1, wf2, bf2, g2, b2,
                  w1a, b1u, w1b, hn_out, a_out, b_out):
    agg = (on0[...] + on1[...]) / (den0[...] + den1[...] + 1e-16)
    out_node = _dot(agg, wo[...]) + bo[...]
    h1 = _ln(hnode[...] + out_node, g1[...], b1[...])
    ff = _dot(_gelu(_dot(h1, wf1[...]) + bf1[...]), wf2[...]) + bf2[...]
    hn = _ln(h1 + ff, g2[...], b2[...])
    hn_out[...] = hn
    a_out[...] = _dot(hn, w1a[...]) + b1u[...]
    b_out[...] = _dot(hn, w1b[...])


def _tc_edge_body(an, bn, he, w1c, w2, b2, ge, be, he_out):
    t = an[...] + bn[...] + _dot(he[...], w1c[...])
    hen = _dot(_gelu(t), w2[...]) + b2[...]
    he_out[...] = _ln(he[...] + hen, ge[...], be[...])


NPAIR = (NCH - 1) // 2


def _make_gather(mesh, use_dst):
    n = len(use_dst)

    @functools.partial(
        pl.kernel,
        mesh=mesh,
        out_type=[jax.ShapeDtypeStruct((E, D), jnp.float32)] * n,
        scratch_types=(
            [pltpu.VMEM((CB,), jnp.int32)] * 4
            + [pltpu.VMEM((CB, D), jnp.float32)] * (2 * n)
            + [pltpu.SemaphoreType.DMA] * 4
        ),
    )
    def gather(*refs):
        tabs = refs[:n]
        src_hbm, dst_hbm = refs[n], refs[n + 1]
        outs = refs[n + 2:2 * n + 2]
        scr = refs[2 * n + 2:]
        idx = (scr[0:2], scr[2:4])
        bufs = (scr[4:4 + n], scr[4 + n:4 + 2 * n])
        sem_g = scr[4 + 2 * n:6 + 2 * n]
        sem_w = scr[6 + 2 * n:8 + 2 * n]

        wid = lax.axis_index("s") * NC + lax.axis_index("c")
        base = wid * PER_W

        def gidx(slot, t):
            return idx[slot][1] if use_dst[t] else idx[slot][0]

        def fire(c, slot):
            off = base + c * CB
            pltpu.sync_copy(src_hbm.at[pl.ds(off, CB)], idx[slot][0])
            pltpu.sync_copy(dst_hbm.at[pl.ds(off, CB)], idx[slot][1])
            for t in range(n):
                pltpu.async_copy(tabs[t].at[gidx(slot, t)], bufs[slot][t],
                                 sem_g[slot])

        def wait_g(slot):
            for t in range(n):
                pltpu.make_async_copy(tabs[t].at[gidx(slot, t)],
                                      bufs[slot][t], sem_g[slot]).wait()

        def fire_w(c, slot):
            off = base + c * CB
            for t in range(n):
                pltpu.async_copy(bufs[slot][t], outs[t].at[pl.ds(off, CB)],
                                 sem_w[slot])

        def wait_w(c, slot):
            off = base + c * CB
            for t in range(n):
                pltpu.make_async_copy(bufs[slot][t],
                                      outs[t].at[pl.ds(off, CB)],
                                      sem_w[slot]).wait()

        fire(0, 0)
        fire(1, 1)

        def body(j, carry):
            c0 = 2 * j
            wait_g(0)
            fire_w(c0, 0)
            wait_g(1)
            fire_w(c0 + 1, 1)
            wait_w(c0, 0)
            fire(c0 + 2, 0)
            wait_w(c0 + 1, 1)

            @pl.when(j < NPAIR - 1)
            def _prefetch():
                fire(c0 + 3, 1)

            return carry

        lax.fori_loop(0, NPAIR, body, 0)
        wait_g(0)
        fire_w(NCH - 1, 0)
        wait_w(NCH - 1, 0)

    return gather


@functools.cache
def _sc_kernels():
    mesh = plsc.VectorSubcoreMesh(core_axis_name="c", subcore_axis_name="s")

    gather_qkv = _make_gather(mesh, (True, False, False))
    gather_ab = _make_gather(mesh, (False, True))

    @functools.partial(
        pl.kernel,
        mesh=mesh,
        out_type=[jax.ShapeDtypeStruct((NC, NPAD, D), jnp.float32)] * 2,
        scratch_types=[
            pltpu.VMEM((CB,), jnp.int32),
            pltpu.VMEM((CB,), jnp.int32),
            pltpu.VMEM((CB, D), jnp.float32),
            pltpu.VMEM((CB, D), jnp.float32),
            pltpu.VMEM_SHARED((NPAD, D), jnp.float32),
            pltpu.SemaphoreType.DMA,
            pltpu.SemaphoreType.DMA,
        ],
    )
    def scatter(dst_hbm, msg_hbm, exx_hbm, zero_hbm, on_out, den_out,
                idx0, idx1, buf0, buf1, sh, sem0, sem1):
        cid = lax.axis_index("c")
        sid = lax.axis_index("s")
        wid = sid * NC + cid
        r0 = sid * ROWS_PER_SUB
        base = wid * PER_W
        idxs = (idx0, idx1)
        bufs = (buf0, buf1)
        sems = (sem0, sem1)

        def phase(payload_hbm, out_hbm):
            pltpu.sync_copy(zero_hbm.at[pl.ds(r0, ROWS_PER_SUB)],
                            sh.at[pl.ds(r0, ROWS_PER_SUB)])
            plsc.subcore_barrier()

            def fire_l(c, slot):
                off = base + c * CB
                pltpu.async_copy(dst_hbm.at[pl.ds(off, CB)], idxs[slot],
                                 sems[slot])
                pltpu.async_copy(payload_hbm.at[pl.ds(off, CB)], bufs[slot],
                                 sems[slot])

            def wait_l(c, slot):
                off = base + c * CB
                pltpu.make_async_copy(dst_hbm.at[pl.ds(off, CB)], idxs[slot],
                                      sems[slot]).wait()
                pltpu.make_async_copy(payload_hbm.at[pl.ds(off, CB)],
                                      bufs[slot], sems[slot]).wait()

            fire_l(0, 0)
            fire_l(1, 1)

            def body(j, carry):
                c0 = 2 * j
                wait_l(c0, 0)
                pltpu.sync_copy(bufs[0], sh.at[idxs[0]], add=True)
                fire_l(c0 + 2, 0)
                wait_l(c0 + 1, 1)
                pltpu.sync_copy(bufs[1], sh.at[idxs[1]], add=True)

                @pl.when(j < NPAIR - 1)
                def _prefetch():
                    fire_l(c0 + 3, 1)

                return carry

            lax.fori_loop(0, NPAIR, body, 0)
            wait_l(NCH - 1, 0)
            pltpu.sync_copy(bufs[0], sh.at[idxs[0]], add=True)
            plsc.subcore_barrier()
            pltpu.sync_copy(sh.at[pl.ds(r0, ROWS_PER_SUB)],
                            out_hbm.at[cid, pl.ds(r0, ROWS_PER_SUB)])
            plsc.subcore_barrier()

        phase(msg_hbm, on_out)
        phase(exx_hbm, den_out)

    return gather_qkv, gather_ab, scatter


def kernel(h_node, h_edge, edge_index, params):
    p = params
    _sc_gather_qkv, _sc_gather_ab, _sc_scatter = _sc_kernels()
    ei = edge_index.astype(jnp.int32)
    e_src, e_dst = ei[0], ei[1]
    expm = jnp.asarray(_EXP_NP)
    hs = jnp.asarray(_EXP_NP.T)
    r2 = lambda t: t.reshape(1, -1)

    q, k, v = pl.pallas_call(
        _tc_qkv_body,
        out_shape=[jax.ShapeDtypeStruct((N, D), jnp.float32)] * 3,
    )(h_node, p["q"]["W"], r2(p["q"]["b"]), p["k"]["W"], r2(p["k"]["b"]),
      p["v"]["W"], r2(p["v"]["b"]))

    eb = pl.pallas_call(
        _tc_eb_body,
        grid=(GE,),
        in_specs=[pl.BlockSpec((EB, D), lambda i: (i, 0)),
                  pl.BlockSpec((D, H), lambda i: (0, 0)),
                  pl.BlockSpec((1, H), lambda i: (0, 0))],
        out_specs=pl.BlockSpec((EB, H), lambda i: (i, 0)),
        out_shape=jax.ShapeDtypeStruct((E, H), jnp.float32),
    )(h_edge, p["eb"]["W"], r2(p["eb"]["b"]))

    qd, ks, vs = _sc_gather_qkv(q, k, v, e_src, e_dst)

    comb = pl.pallas_call(
        _tc_msg_body,
        grid=(GE,),
        in_specs=[pl.BlockSpec((EB, D), lambda i: (i, 0)),
                  pl.BlockSpec((EB, D), lambda i: (i, 0)),
                  pl.BlockSpec((EB, D), lambda i: (i, 0)),
                  pl.BlockSpec((EB, H), lambda i: (i, 0)),
                  pl.BlockSpec((D, H), lambda i: (0, 0)),
                  pl.BlockSpec((H, D), lambda i: (0, 0))],
        out_specs=[pl.BlockSpec((EB, D), lambda i: (i, 0))] * 2,
        out_shape=[jax.ShapeDtypeStruct((E, D), jnp.float32)] * 2,
    )(qd, ks, vs, eb, hs, expm)
    msg, exx = comb

    zero = jnp.zeros((NPAD, D), jnp.float32)
    on_p, den_p = _sc_scatter(e_dst, msg, exx, zero)

    wspec = lambda shp: pl.BlockSpec(shp, lambda i: (0, 0))
    hn, a_tab, b_tab = pl.pallas_call(
        _tc_node_body,
        grid=(GN,),
        in_specs=[pl.BlockSpec((NB, D), lambda i: (i, 0))] * 5 +
                 [wspec((D, D)), wspec((1, D)),
                  wspec((1, D)), wspec((1, D)),
                  wspec((D, 2 * D)), wspec((1, 2 * D)),
                  wspec((2 * D, D)), wspec((1, D)),
                  wspec((1, D)), wspec((1, D)),
                  wspec((D, D)), wspec((1, D)), wspec((D, D))],
        out_specs=[pl.BlockSpec((NB, D), lambda i: (i, 0))] * 3,
        out_shape=[jax.ShapeDtypeStruct((N, D), jnp.float32)] * 3,
    )(h_node, on_p[0], on_p[1], den_p[0], den_p[1],
      p["o"]["W"], r2(p["o"]["b"]),
      r2(p["ln1"]["g"]), r2(p["ln1"]["b"]),
      p["ffn1"]["W"], r2(p["ffn1"]["b"]),
      p["ffn2"]["W"], r2(p["ffn2"]["b"]),
      r2(p["ln2"]["g"]), r2(p["ln2"]["b"]),
      p["eu1"]["W"][:D], r2(p["eu1"]["b"]), p["eu1"]["W"][D:2 * D])

    an, bn = _sc_gather_ab(a_tab, b_tab, e_src, e_dst)

    he = pl.pallas_call(
        _tc_edge_body,
        grid=(GE,),
        in_specs=[pl.BlockSpec((EB, D), lambda i: (i, 0)),
                  pl.BlockSpec((EB, D), lambda i: (i, 0)),
                  pl.BlockSpec((EB, D), lambda i: (i, 0)),
                  wspec((D, D)), wspec((D, D)), wspec((1, D)),
                  wspec((1, D)), wspec((1, D))],
        out_specs=pl.BlockSpec((EB, D), lambda i: (i, 0)),
        out_shape=jax.ShapeDtypeStruct((E, D), jnp.float32),
    )(an, bn, h_edge, p["eu1"]["W"][2 * D:], p["eu2"]["W"], r2(p["eu2"]["b"]),
      r2(p["lne"]["g"]), r2(p["lne"]["b"]))

    return hn, he
